# Initial kernel scaffold; baseline (speedup 1.0000x reference)
import jax
import jax.numpy as jnp
from jax.experimental import pallas as pl


def kernel(features, edge_index, W1, b1, W2, b2, Wr, br):
    raise NotImplementedError("write your pallas kernel here")


# speedup vs baseline: 5.6313x; 5.6313x over previous
import functools

import jax
import jax.numpy as jnp
from jax import lax
from jax.experimental import pallas as pl
from jax.experimental.pallas import tpu as pltpu
from jax.experimental.pallas import tpu_sc as plsc

N_NODES = 10000
N_EDGES = 320000
D_IN = 128
D_HID = 128
D_OUT = 16

NPAD = 10240
PADID = N_NODES
EPAD = 327680
GSZ = 64
NGRP = EPAD // GSZ

NC, NS = 2, 16
ROWS_PER_TILE = NPAD // NS
GRP_W = NGRP // (NC * NS)
EDGES_W = EPAD // (NC * NS)
STEPS = EDGES_W // 16


@functools.cache
def _sc_mesh():
    return plsc.VectorSubcoreMesh(core_axis_name="c", subcore_axis_name="s",
                                  num_cores=NC, num_subcores=NS)


def _deg_body(srcp, dstp, ones_h, z1d, dego_out, degi_out,
              dego_sh, degi_sh, src_v, dst_v, ones_v):
    c = lax.axis_index("c")
    s = lax.axis_index("s")
    w = c * NS + s
    r0 = s * ROWS_PER_TILE
    pltpu.sync_copy(z1d.at[pl.ds(r0, ROWS_PER_TILE)],
                    dego_sh.at[pl.ds(r0, ROWS_PER_TILE)])
    pltpu.sync_copy(z1d.at[pl.ds(r0, ROWS_PER_TILE)],
                    degi_sh.at[pl.ds(r0, ROWS_PER_TILE)])
    pltpu.sync_copy(ones_h, ones_v)
    pltpu.sync_copy(srcp.at[pl.ds(w * GRP_W, GRP_W)], src_v)
    pltpu.sync_copy(dstp.at[pl.ds(w * GRP_W, GRP_W)], dst_v)
    plsc.subcore_barrier()

    def body(j, carry):
        pltpu.sync_copy(ones_v, dego_sh.at[src_v.at[j]], add=True)
        pltpu.sync_copy(ones_v, degi_sh.at[dst_v.at[j]], add=True)
        return carry

    lax.fori_loop(0, GRP_W, body, 0)
    plsc.subcore_barrier()

    @pl.when(s == 0)
    def _():
        pltpu.sync_copy(dego_sh, dego_out.at[c])
        pltpu.sync_copy(degi_sh, degi_out.at[c])


@functools.cache
def _deg_call():
    return pl.kernel(
        _deg_body,
        out_type=(
            jax.ShapeDtypeStruct((NC, NPAD), jnp.float32),
            jax.ShapeDtypeStruct((NC, NPAD), jnp.float32),
        ),
        mesh=_sc_mesh(),
        scratch_types=[
            pltpu.VMEM_SHARED((NPAD,), jnp.float32),
            pltpu.VMEM_SHARED((NPAD,), jnp.float32),
            pltpu.VMEM((GRP_W, GSZ), jnp.int32),
            pltpu.VMEM((GRP_W, GSZ), jnp.int32),
            pltpu.VMEM((GSZ,), jnp.float32),
        ],
    )


def _prep_body(x_ref, w1_ref, dego_ref, degi_ref, g_ref, ns_ref, nd_ref):
    dego = dego_ref[0] + dego_ref[1]
    degi = degi_ref[0] + degi_ref[1]
    ns = lax.rsqrt(jnp.where(dego > 0.0, dego, 1.0))
    nd = lax.rsqrt(jnp.where(degi > 0.0, degi, 1.0))
    h = jnp.dot(x_ref[...], w1_ref[...], preferred_element_type=jnp.float32)
    g_ref[...] = h * ns
    ns_ref[...] = ns
    nd_ref[...] = nd


_prep_call = pl.pallas_call(
    _prep_body,
    out_shape=(
        jax.ShapeDtypeStruct((NPAD, D_HID), jnp.float32),
        jax.ShapeDtypeStruct((NPAD, 1), jnp.float32),
        jax.ShapeDtypeStruct((NPAD, 1), jnp.float32),
    ),
)


def _agg_body(g, srcp, dstp, ndv, z2d, z1d,
              agg_parts, s_parts,
              acc_sh, s_sh, src_v, dst_v, rows_v, nvals_v, gsem):
    c = lax.axis_index("c")
    s = lax.axis_index("s")
    w = c * NS + s
    r0 = s * ROWS_PER_TILE
    pltpu.sync_copy(z2d.at[pl.ds(r0, ROWS_PER_TILE)],
                    acc_sh.at[pl.ds(r0, ROWS_PER_TILE)])

    @pl.when(s == 0)
    def _():
        pltpu.sync_copy(z1d, s_sh)

    pltpu.sync_copy(srcp.at[pl.ds(w * EDGES_W, EDGES_W)], src_v)
    pltpu.sync_copy(dstp.at[pl.ds(w * EDGES_W, EDGES_W)], dst_v)
    plsc.subcore_barrier()

    pltpu.async_copy(g.at[src_v[pl.ds(0, 16)]], rows_v.at[0], gsem)

    def body(i, carry):
        cur = lax.rem(i, 2)
        pltpu.make_async_copy(g.at[src_v[pl.ds(i * 16, 16)]],
                              rows_v.at[cur], gsem).wait()

        @pl.when(i < STEPS - 1)
        def _():
            pltpu.async_copy(g.at[src_v[pl.ds((i + 1) * 16, 16)]],
                             rows_v.at[lax.rem(i + 1, 2)], gsem)

        pltpu.sync_copy(rows_v.at[cur],
                        acc_sh.at[dst_v[pl.ds(i * 16, 16)]], add=True)
        return carry

    lax.fori_loop(0, STEPS, body, 0)

    def sbody(i, carry):
        pltpu.sync_copy(ndv.at[dst_v[pl.ds(i * 16, 16)]], nvals_v)
        pltpu.sync_copy(nvals_v, s_sh.at[src_v[pl.ds(i * 16, 16)]], add=True)
        return carry

    lax.fori_loop(0, STEPS, sbody, 0)
    plsc.subcore_barrier()

    @pl.when(c == 0)
    def _():
        pltpu.sync_copy(acc_sh.at[pl.ds(r0, ROWS_PER_TILE)],
                        agg_parts.at[0].at[pl.ds(r0, ROWS_PER_TILE)])

        @pl.when(s == 0)
        def _():
            pltpu.sync_copy(s_sh, s_parts.at[0])

    @pl.when(c == 1)
    def _():
        pltpu.sync_copy(acc_sh.at[pl.ds(r0, ROWS_PER_TILE)],
                        agg_parts.at[1].at[pl.ds(r0, ROWS_PER_TILE)])

        @pl.when(s == 0)
        def _():
            pltpu.sync_copy(s_sh, s_parts.at[1])


@functools.cache
def _agg_call():
    return pl.kernel(
        _agg_body,
        out_type=(
            jax.ShapeDtypeStruct((NC, NPAD, D_HID), jnp.float32),
            jax.ShapeDtypeStruct((NC, NPAD), jnp.float32),
        ),
        mesh=_sc_mesh(),
        scratch_types=[
            pltpu.VMEM_SHARED((NPAD, D_HID), jnp.float32),
            pltpu.VMEM_SHARED((NPAD,), jnp.float32),
            pltpu.VMEM((EDGES_W,), jnp.int32),
            pltpu.VMEM((EDGES_W,), jnp.int32),
            pltpu.VMEM((2, 16, D_HID), jnp.float32),
            pltpu.VMEM((16,), jnp.float32),
            pltpu.SemaphoreType.DMA,
        ],
    )


def _final_body(ap, ns, nd, sp, b1, w2, b2, wr, br, out_ref):
    agg = ap[0] + ap[1]
    x1 = jnp.maximum(agg * nd[...] + b1[...][None, :], 0.0)
    cvec = ns[...] * (sp[0] + sp[1])
    v = jnp.sum(x1 * cvec, axis=0, keepdims=True) * (1.0 / N_NODES)
    h2 = jnp.dot(v, w2[...], preferred_element_type=jnp.float32)
    h2 = h2 + b2[...][None, :]
    out = jnp.dot(h2, wr[...], preferred_element_type=jnp.float32)
    out_ref[...] = out + br[...][None, :]


_final_call = pl.pallas_call(
    _final_body,
    out_shape=jax.ShapeDtypeStruct((1, D_OUT), jnp.float32),
)


def kernel(features, edge_index, W1, b1, W2, b2, Wr, br):
    src = edge_index[0].astype(jnp.int32)
    dst = edge_index[1].astype(jnp.int32)
    pad = jnp.full((EPAD - N_EDGES,), PADID, jnp.int32)
    src1 = jnp.concatenate([src, pad])
    dst1 = jnp.concatenate([dst, pad])
    srcp = src1.reshape(NGRP, GSZ)
    dstp = dst1.reshape(NGRP, GSZ)
    xpad = jnp.pad(features, ((0, NPAD - N_NODES), (0, 0)))
    z1 = jnp.zeros((NPAD,), jnp.float32)
    z2 = jnp.zeros((NPAD, D_HID), jnp.float32)
    onesg = jnp.ones((GSZ,), jnp.float32)

    dego_p, degi_p = _deg_call()(srcp, dstp, onesg, z1)
    g, ns2, nd2 = _prep_call(xpad, W1, dego_p[:, :, None], degi_p[:, :, None])
    agg_p, s_p = _agg_call()(g, src1, dst1, nd2.reshape(NPAD), z2, z1)
    out = _final_call(agg_p[:, :N_NODES], ns2[:N_NODES], nd2[:N_NODES],
                      s_p[:, :N_NODES, None], b1, W2, b2, Wr, br)
    return out


# speedup vs baseline: 11.2256x; 1.9934x over previous
import functools

import jax
import jax.numpy as jnp
from jax import lax
from jax.experimental import pallas as pl
from jax.experimental.pallas import tpu as pltpu
from jax.experimental.pallas import tpu_sc as plsc

N_NODES = 10000
N_EDGES = 320000
D_IN = 128
D_HID = 128
D_OUT = 16

NPAD = 10240
PADID = N_NODES
EPAD = 327680
GSZ = 64
NGRP = EPAD // GSZ

NC, NS = 2, 16
ROWS_PER_TILE = NPAD // NS
GRP_W = NGRP // (NC * NS)
EDGES_W = EPAD // (NC * NS)
STEPS = EDGES_W // 16


@functools.cache
def _sc_mesh():
    return plsc.VectorSubcoreMesh(core_axis_name="c", subcore_axis_name="s",
                                  num_cores=NC, num_subcores=NS)


def _deg_body(srcp, dstp, ones_h, z1d, dego_out, degi_out,
              dego_sh, degi_sh, src_v, dst_v, ones_v):
    c = lax.axis_index("c")
    s = lax.axis_index("s")
    w = c * NS + s
    r0 = s * ROWS_PER_TILE
    pltpu.sync_copy(z1d.at[pl.ds(r0, ROWS_PER_TILE)],
                    dego_sh.at[pl.ds(r0, ROWS_PER_TILE)])
    pltpu.sync_copy(z1d.at[pl.ds(r0, ROWS_PER_TILE)],
                    degi_sh.at[pl.ds(r0, ROWS_PER_TILE)])
    pltpu.sync_copy(ones_h, ones_v)
    pltpu.sync_copy(srcp.at[pl.ds(w * GRP_W, GRP_W)], src_v)
    pltpu.sync_copy(dstp.at[pl.ds(w * GRP_W, GRP_W)], dst_v)
    plsc.subcore_barrier()

    def body(j, carry):
        pltpu.sync_copy(ones_v, dego_sh.at[src_v.at[j]], add=True)
        pltpu.sync_copy(ones_v, degi_sh.at[dst_v.at[j]], add=True)
        return carry

    lax.fori_loop(0, GRP_W, body, 0)
    plsc.subcore_barrier()

    @pl.when(s == 0)
    def _():
        pltpu.sync_copy(dego_sh, dego_out.at[c])
        pltpu.sync_copy(degi_sh, degi_out.at[c])


@functools.cache
def _deg_call():
    return pl.kernel(
        _deg_body,
        out_type=(
            jax.ShapeDtypeStruct((NC, NPAD), jnp.float32),
            jax.ShapeDtypeStruct((NC, NPAD), jnp.float32),
        ),
        mesh=_sc_mesh(),
        scratch_types=[
            pltpu.VMEM_SHARED((NPAD,), jnp.float32),
            pltpu.VMEM_SHARED((NPAD,), jnp.float32),
            pltpu.VMEM((GRP_W, GSZ), jnp.int32),
            pltpu.VMEM((GRP_W, GSZ), jnp.int32),
            pltpu.VMEM((GSZ,), jnp.float32),
        ],
    )


def _prep_body(x_ref, w1_ref, dego_ref, degi_ref, g_ref, ns_ref, nd_ref):
    dego = dego_ref[0] + dego_ref[1]
    degi = degi_ref[0] + degi_ref[1]
    ns = lax.rsqrt(jnp.where(dego > 0.0, dego, 1.0))
    nd = lax.rsqrt(jnp.where(degi > 0.0, degi, 1.0))
    h = jnp.dot(x_ref[...], w1_ref[...], preferred_element_type=jnp.float32)
    g_ref[...] = h * ns
    ns_ref[...] = ns
    nd_ref[...] = nd


_prep_call = pl.pallas_call(
    _prep_body,
    out_shape=(
        jax.ShapeDtypeStruct((NPAD, D_HID), jnp.float32),
        jax.ShapeDtypeStruct((NPAD, 1), jnp.float32),
        jax.ShapeDtypeStruct((NPAD, 1), jnp.float32),
    ),
)


RING = 8


def _agg_body(g, srcp, dstp, ndv, z2d, z1d,
              agg_parts, s_parts,
              acc_sh, s_sh, src_v, dst_v, rows_v, svals_v,
              gsem, g2sem):
    c = lax.axis_index("c")
    s = lax.axis_index("s")
    w = c * NS + s
    r0 = s * ROWS_PER_TILE
    pltpu.sync_copy(z2d.at[pl.ds(r0, ROWS_PER_TILE)],
                    acc_sh.at[pl.ds(r0, ROWS_PER_TILE)])

    @pl.when(s == 0)
    def _():
        pltpu.sync_copy(z1d, s_sh)

    pltpu.sync_copy(srcp.at[pl.ds(w * EDGES_W, EDGES_W)], src_v)
    pltpu.sync_copy(dstp.at[pl.ds(w * EDGES_W, EDGES_W)], dst_v)
    plsc.subcore_barrier()

    def fire_gathers(i, b):
        si = src_v[pl.ds(i * 16, 16)]
        di = dst_v[pl.ds(i * 16, 16)]
        pltpu.async_copy(g.at[si], rows_v.at[b], gsem.at[b])
        pltpu.async_copy(ndv.at[di], svals_v.at[b], g2sem.at[b])

    for b in range(RING):
        fire_gathers(b, b)

    nsuper = STEPS // RING

    def body(sc, carry):
        for b in range(RING):
            i = sc * RING + b
            si = src_v[pl.ds(i * 16, 16)]
            di = dst_v[pl.ds(i * 16, 16)]
            pltpu.make_async_copy(g.at[si], rows_v.at[b], gsem.at[b]).wait()
            pltpu.make_async_copy(ndv.at[di], svals_v.at[b],
                                  g2sem.at[b]).wait()
            pltpu.sync_copy(rows_v.at[b], acc_sh.at[di], add=True)
            pltpu.sync_copy(svals_v.at[b], s_sh.at[si], add=True)

            @pl.when(sc < nsuper - 1)
            def _():
                fire_gathers(i + RING, b)

        return carry

    lax.fori_loop(0, nsuper, body, 0)
    plsc.subcore_barrier()

    @pl.when(c == 0)
    def _():
        pltpu.sync_copy(acc_sh.at[pl.ds(r0, ROWS_PER_TILE)],
                        agg_parts.at[0].at[pl.ds(r0, ROWS_PER_TILE)])

        @pl.when(s == 0)
        def _():
            pltpu.sync_copy(s_sh, s_parts.at[0])

    @pl.when(c == 1)
    def _():
        pltpu.sync_copy(acc_sh.at[pl.ds(r0, ROWS_PER_TILE)],
                        agg_parts.at[1].at[pl.ds(r0, ROWS_PER_TILE)])

        @pl.when(s == 0)
        def _():
            pltpu.sync_copy(s_sh, s_parts.at[1])


@functools.cache
def _agg_call():
    return pl.kernel(
        _agg_body,
        out_type=(
            jax.ShapeDtypeStruct((NC, NPAD, D_HID), jnp.float32),
            jax.ShapeDtypeStruct((NC, NPAD), jnp.float32),
        ),
        mesh=_sc_mesh(),
        scratch_types=[
            pltpu.VMEM_SHARED((NPAD, D_HID), jnp.float32),
            pltpu.VMEM_SHARED((NPAD,), jnp.float32),
            pltpu.VMEM((EDGES_W,), jnp.int32),
            pltpu.VMEM((EDGES_W,), jnp.int32),
            pltpu.VMEM((RING, 16, D_HID), jnp.float32),
            pltpu.VMEM((RING, 16), jnp.float32),
            pltpu.SemaphoreType.DMA((RING,)),
            pltpu.SemaphoreType.DMA((RING,)),
        ],
    )


def _final_body(ap, ns, nd, sp, b1, w2, b2, wr, br, out_ref):
    agg = ap[0] + ap[1]
    x1 = jnp.maximum(agg * nd[...] + b1[...][None, :], 0.0)
    cvec = ns[...] * (sp[0] + sp[1])
    v = jnp.sum(x1 * cvec, axis=0, keepdims=True) * (1.0 / N_NODES)
    h2 = jnp.dot(v, w2[...], preferred_element_type=jnp.float32)
    h2 = h2 + b2[...][None, :]
    out = jnp.dot(h2, wr[...], preferred_element_type=jnp.float32)
    out_ref[...] = out + br[...][None, :]


_final_call = pl.pallas_call(
    _final_body,
    out_shape=jax.ShapeDtypeStruct((1, D_OUT), jnp.float32),
)


def kernel(features, edge_index, W1, b1, W2, b2, Wr, br):
    src = edge_index[0].astype(jnp.int32)
    dst = edge_index[1].astype(jnp.int32)
    pad = jnp.full((EPAD - N_EDGES,), PADID, jnp.int32)
    src1 = jnp.concatenate([src, pad])
    dst1 = jnp.concatenate([dst, pad])
    srcp = src1.reshape(NGRP, GSZ)
    dstp = dst1.reshape(NGRP, GSZ)
    xpad = jnp.pad(features, ((0, NPAD - N_NODES), (0, 0)))
    z1 = jnp.zeros((NPAD,), jnp.float32)
    z2 = jnp.zeros((NPAD, D_HID), jnp.float32)
    onesg = jnp.ones((GSZ,), jnp.float32)

    dego_p, degi_p = _deg_call()(srcp, dstp, onesg, z1)
    g, ns2, nd2 = _prep_call(xpad, W1, dego_p[:, :, None], degi_p[:, :, None])
    agg_p, s_p = _agg_call()(g, src1, dst1, nd2.reshape(NPAD), z2, z1)
    out = _final_call(agg_p[:, :N_NODES], ns2[:N_NODES], nd2[:N_NODES],
                      s_p[:, :N_NODES, None], b1, W2, b2, Wr, br)
    return out


# speedup vs baseline: 22.4813x; 2.0027x over previous
import functools

import jax
import jax.numpy as jnp
from jax import lax
from jax.experimental import pallas as pl
from jax.experimental.pallas import tpu as pltpu
from jax.experimental.pallas import tpu_sc as plsc

N_NODES = 10000
N_EDGES = 320000
D_IN = 128
D_HID = 128
D_OUT = 16

NPAD = 10240
PADID = N_NODES
EPAD = 327680
GSZ = 64
NGRP = EPAD // GSZ

NC, NS = 2, 16
ROWS_PER_TILE = NPAD // NS
GRP_W = NGRP // (NC * NS)
EDGES_W = EPAD // (NC * NS)
STEPS = EDGES_W // 16


@functools.cache
def _sc_mesh():
    return plsc.VectorSubcoreMesh(core_axis_name="c", subcore_axis_name="s",
                                  num_cores=NC, num_subcores=NS)


def _deg_body(srcp, dstp, ones_h, z1d, dego_out, degi_out,
              dego_sh, degi_sh, src_v, dst_v, ones_v):
    c = lax.axis_index("c")
    s = lax.axis_index("s")
    w = c * NS + s
    r0 = s * ROWS_PER_TILE
    pltpu.sync_copy(z1d.at[pl.ds(r0, ROWS_PER_TILE)],
                    dego_sh.at[pl.ds(r0, ROWS_PER_TILE)])
    pltpu.sync_copy(z1d.at[pl.ds(r0, ROWS_PER_TILE)],
                    degi_sh.at[pl.ds(r0, ROWS_PER_TILE)])
    pltpu.sync_copy(ones_h, ones_v)
    pltpu.sync_copy(srcp.at[pl.ds(w * GRP_W, GRP_W)], src_v)
    pltpu.sync_copy(dstp.at[pl.ds(w * GRP_W, GRP_W)], dst_v)
    plsc.subcore_barrier()

    def body(j, carry):
        pltpu.sync_copy(ones_v, dego_sh.at[src_v.at[j]], add=True)
        pltpu.sync_copy(ones_v, degi_sh.at[dst_v.at[j]], add=True)
        return carry

    lax.fori_loop(0, GRP_W, body, 0)
    plsc.subcore_barrier()

    @pl.when(s == 0)
    def _():
        pltpu.sync_copy(dego_sh, dego_out.at[c])
        pltpu.sync_copy(degi_sh, degi_out.at[c])


@functools.cache
def _deg_call():
    return pl.kernel(
        _deg_body,
        out_type=(
            jax.ShapeDtypeStruct((NC, NPAD), jnp.float32),
            jax.ShapeDtypeStruct((NC, NPAD), jnp.float32),
        ),
        mesh=_sc_mesh(),
        scratch_types=[
            pltpu.VMEM_SHARED((NPAD,), jnp.float32),
            pltpu.VMEM_SHARED((NPAD,), jnp.float32),
            pltpu.VMEM((GRP_W, GSZ), jnp.int32),
            pltpu.VMEM((GRP_W, GSZ), jnp.int32),
            pltpu.VMEM((GSZ,), jnp.float32),
        ],
    )


def _prep_body(x_ref, w1_ref, dego_ref, degi_ref, g_ref, ns_ref, nd_ref):
    dego = dego_ref[0] + dego_ref[1]
    degi = degi_ref[0] + degi_ref[1]
    ns = lax.rsqrt(jnp.where(dego > 0.0, dego, 1.0))
    nd = lax.rsqrt(jnp.where(degi > 0.0, degi, 1.0))
    h = jnp.dot(x_ref[...], w1_ref[...], preferred_element_type=jnp.float32)
    g_ref[...] = h * ns
    ns_ref[...] = ns
    nd_ref[...] = nd


_prep_call = pl.pallas_call(
    _prep_body,
    out_shape=(
        jax.ShapeDtypeStruct((NPAD, D_HID), jnp.float32),
        jax.ShapeDtypeStruct((NPAD, 1), jnp.float32),
        jax.ShapeDtypeStruct((NPAD, 1), jnp.float32),
    ),
)


RING = 8


def _agg_body(g, srcp, dstp, ndv, z2d, z1d,
              agg_parts, s_parts,
              acc_sh, s_sh, src_v, dst_v, rows_v, svals_v,
              gsem, g2sem):
    c = lax.axis_index("c")
    s = lax.axis_index("s")
    w = c * NS + s
    r0 = s * ROWS_PER_TILE
    pltpu.sync_copy(z2d.at[pl.ds(r0, ROWS_PER_TILE)],
                    acc_sh.at[pl.ds(r0, ROWS_PER_TILE)])

    @pl.when(s == 0)
    def _():
        pltpu.sync_copy(z1d, s_sh)

    pltpu.sync_copy(srcp.at[pl.ds(w * EDGES_W, EDGES_W)], src_v)
    pltpu.sync_copy(dstp.at[pl.ds(w * EDGES_W, EDGES_W)], dst_v)
    plsc.subcore_barrier()

    def fire_gathers(i, b):
        si = src_v[pl.ds(i * 16, 16)]
        di = dst_v[pl.ds(i * 16, 16)]
        pltpu.async_copy(g.at[si], rows_v.at[b], gsem.at[b])
        pltpu.async_copy(ndv.at[di], svals_v.at[b], g2sem.at[b])

    for b in range(RING):
        fire_gathers(b, b)

    nsuper = STEPS // RING

    def body(sc, carry):
        for b in range(RING):
            i = sc * RING + b
            si = src_v[pl.ds(i * 16, 16)]
            di = dst_v[pl.ds(i * 16, 16)]
            pltpu.make_async_copy(g.at[si], rows_v.at[b], gsem.at[b]).wait()
            pltpu.make_async_copy(ndv.at[di], svals_v.at[b],
                                  g2sem.at[b]).wait()
            pltpu.sync_copy(rows_v.at[b], acc_sh.at[di], add=True)
            pltpu.sync_copy(svals_v.at[b], s_sh.at[si], add=True)

            @pl.when(sc < nsuper - 1)
            def _():
                fire_gathers(i + RING, b)

        return carry

    lax.fori_loop(0, nsuper, body, 0)
    plsc.subcore_barrier()

    @pl.when(c == 0)
    def _():
        pltpu.sync_copy(acc_sh.at[pl.ds(r0, ROWS_PER_TILE)],
                        agg_parts.at[0].at[pl.ds(r0, ROWS_PER_TILE)])

        @pl.when(s == 0)
        def _():
            pltpu.sync_copy(s_sh, s_parts.at[0])

    @pl.when(c == 1)
    def _():
        pltpu.sync_copy(acc_sh.at[pl.ds(r0, ROWS_PER_TILE)],
                        agg_parts.at[1].at[pl.ds(r0, ROWS_PER_TILE)])

        @pl.when(s == 0)
        def _():
            pltpu.sync_copy(s_sh, s_parts.at[1])


@functools.cache
def _agg_call():
    return pl.kernel(
        _agg_body,
        out_type=(
            jax.ShapeDtypeStruct((NC, NPAD, D_HID), jnp.float32),
            jax.ShapeDtypeStruct((NC, NPAD), jnp.float32),
        ),
        mesh=_sc_mesh(),
        scratch_types=[
            pltpu.VMEM_SHARED((NPAD, D_HID), jnp.float32),
            pltpu.VMEM_SHARED((NPAD,), jnp.float32),
            pltpu.VMEM((EDGES_W,), jnp.int32),
            pltpu.VMEM((EDGES_W,), jnp.int32),
            pltpu.VMEM((RING, 16, D_HID), jnp.float32),
            pltpu.VMEM((RING, 16), jnp.float32),
            pltpu.SemaphoreType.DMA((RING,)),
            pltpu.SemaphoreType.DMA((RING,)),
        ],
    )


def _final_body(ap, ns, nd, sp, b1, w2, b2, wr, br, out_ref):
    agg = ap[0] + ap[1]
    x1 = jnp.maximum(agg * nd[...] + b1[...][None, :], 0.0)
    cvec = ns[...] * (sp[0] + sp[1])
    v = jnp.sum(x1 * cvec, axis=0, keepdims=True) * (1.0 / N_NODES)
    h2 = jnp.dot(v, w2[...], preferred_element_type=jnp.float32)
    h2 = h2 + b2[...][None, :]
    out = jnp.dot(h2, wr[...], preferred_element_type=jnp.float32)
    out_ref[...] = out + br[...][None, :]


_final_call = pl.pallas_call(
    _final_body,
    out_shape=jax.ShapeDtypeStruct((1, D_OUT), jnp.float32),
)


def kernel(features, edge_index, W1, b1, W2, b2, Wr, br):
    src = edge_index[0].astype(jnp.int32)
    dst = edge_index[1].astype(jnp.int32)
    pad = PADID + (jnp.arange(EPAD - N_EDGES, dtype=jnp.int32)
                   % (NPAD - N_NODES))
    src1 = jnp.concatenate([src, pad])
    dst1 = jnp.concatenate([dst, pad])
    srcp = src1.reshape(NGRP, GSZ)
    dstp = dst1.reshape(NGRP, GSZ)
    xpad = jnp.pad(features, ((0, NPAD - N_NODES), (0, 0)))
    z1 = jnp.zeros((NPAD,), jnp.float32)
    z2 = jnp.zeros((NPAD, D_HID), jnp.float32)
    onesg = jnp.ones((GSZ,), jnp.float32)

    dego_p, degi_p = _deg_call()(srcp, dstp, onesg, z1)
    g, ns2, nd2 = _prep_call(xpad, W1, dego_p[:, :, None], degi_p[:, :, None])
    agg_p, s_p = _agg_call()(g, src1, dst1, nd2.reshape(NPAD), z2, z1)
    out = _final_call(agg_p[:, :N_NODES], ns2[:N_NODES], nd2[:N_NODES],
                      s_p[:, :N_NODES, None], b1, W2, b2, Wr, br)
    return out


# speedup vs baseline: 24.5262x; 1.0910x over previous
import functools

import jax
import jax.numpy as jnp
from jax import lax
from jax.experimental import pallas as pl
from jax.experimental.pallas import tpu as pltpu
from jax.experimental.pallas import tpu_sc as plsc

N_NODES = 10000
N_EDGES = 320000
D_IN = 128
D_HID = 128
D_OUT = 16

NPAD = 10240
PADID = N_NODES
EPAD = 327680
GSZ = 128
NGRP = EPAD // GSZ

NC, NS = 2, 16
ROWS_PER_TILE = NPAD // NS
GRP_W = NGRP // (NC * NS)
EDGES_W = EPAD // (NC * NS)
STEPS = EDGES_W // 16


@functools.cache
def _sc_mesh():
    return plsc.VectorSubcoreMesh(core_axis_name="c", subcore_axis_name="s",
                                  num_cores=NC, num_subcores=NS)


def _deg_body(srcp, dstp, ones_h, z1d, dego_out, degi_out,
              dego_sh, degi_sh, src_v, dst_v, ones_v):
    c = lax.axis_index("c")
    s = lax.axis_index("s")
    w = c * NS + s
    r0 = s * ROWS_PER_TILE
    pltpu.sync_copy(z1d.at[pl.ds(r0, ROWS_PER_TILE)],
                    dego_sh.at[pl.ds(r0, ROWS_PER_TILE)])
    pltpu.sync_copy(z1d.at[pl.ds(r0, ROWS_PER_TILE)],
                    degi_sh.at[pl.ds(r0, ROWS_PER_TILE)])
    pltpu.sync_copy(ones_h, ones_v)
    pltpu.sync_copy(srcp.at[pl.ds(w * GRP_W, GRP_W)], src_v)
    pltpu.sync_copy(dstp.at[pl.ds(w * GRP_W, GRP_W)], dst_v)
    plsc.subcore_barrier()

    def body(j, carry):
        pltpu.sync_copy(ones_v, dego_sh.at[src_v.at[j]], add=True)
        pltpu.sync_copy(ones_v, degi_sh.at[dst_v.at[j]], add=True)
        return carry

    lax.fori_loop(0, GRP_W, body, 0)
    plsc.subcore_barrier()

    @pl.when(s == 0)
    def _():
        pltpu.sync_copy(dego_sh, dego_out.at[c])
        pltpu.sync_copy(degi_sh, degi_out.at[c])


@functools.cache
def _deg_call():
    return pl.kernel(
        _deg_body,
        out_type=(
            jax.ShapeDtypeStruct((NC, NPAD), jnp.float32),
            jax.ShapeDtypeStruct((NC, NPAD), jnp.float32),
        ),
        mesh=_sc_mesh(),
        scratch_types=[
            pltpu.VMEM_SHARED((NPAD,), jnp.float32),
            pltpu.VMEM_SHARED((NPAD,), jnp.float32),
            pltpu.VMEM((GRP_W, GSZ), jnp.int32),
            pltpu.VMEM((GRP_W, GSZ), jnp.int32),
            pltpu.VMEM((GSZ,), jnp.float32),
        ],
    )


def _prep_body(x_ref, w1_ref, dego_ref, degi_ref, g_ref, ns_ref, nd_ref):
    dego = dego_ref[0] + dego_ref[1]
    degi = degi_ref[0] + degi_ref[1]
    ns = lax.rsqrt(jnp.where(dego > 0.0, dego, 1.0))
    nd = lax.rsqrt(jnp.where(degi > 0.0, degi, 1.0))
    h = jnp.dot(x_ref[...], w1_ref[...], preferred_element_type=jnp.float32)
    g_ref[pl.ds(0, N_NODES), :] = h * ns[:N_NODES]
    g_ref[pl.ds(N_NODES, NPAD - N_NODES), :] = jnp.zeros(
        (NPAD - N_NODES, D_HID), jnp.float32)
    ns_ref[...] = ns
    nd_ref[...] = nd


_prep_call = pl.pallas_call(
    _prep_body,
    out_shape=(
        jax.ShapeDtypeStruct((NPAD, D_HID), jnp.float32),
        jax.ShapeDtypeStruct((NPAD, 1), jnp.float32),
        jax.ShapeDtypeStruct((NPAD, 1), jnp.float32),
    ),
)


RING = 8


def _agg_body(g, srcp, dstp, ndv, z2d, z1d,
              agg_parts, s_parts,
              acc_sh, s_sh, src_v, dst_v, rows_v, svals_v,
              gsem, g2sem):
    c = lax.axis_index("c")
    s = lax.axis_index("s")
    w = c * NS + s
    r0 = s * ROWS_PER_TILE
    pltpu.sync_copy(z2d.at[pl.ds(r0, ROWS_PER_TILE)],
                    acc_sh.at[pl.ds(r0, ROWS_PER_TILE)])

    @pl.when(s == 0)
    def _():
        pltpu.sync_copy(z1d, s_sh)

    pltpu.sync_copy(srcp.at[pl.ds(w * EDGES_W, EDGES_W)], src_v)
    pltpu.sync_copy(dstp.at[pl.ds(w * EDGES_W, EDGES_W)], dst_v)
    plsc.subcore_barrier()

    def fire_gather(i, b):
        si = src_v[pl.ds(i * 16, 16)]
        di = dst_v[pl.ds(i * 16, 16)]
        pltpu.async_copy(g.at[si], rows_v.at[b], gsem.at[b])
        pltpu.async_copy(ndv.at[di], svals_v.at[pl.ds(b * 16, 16)],
                         g2sem.at[b])

    for b in range(RING):
        fire_gather(b, b)

    nsuper = STEPS // RING

    def body(sc, carry):
        for b in range(RING):
            i = sc * RING + b
            si = src_v[pl.ds(i * 16, 16)]
            di = dst_v[pl.ds(i * 16, 16)]
            pltpu.make_async_copy(g.at[si], rows_v.at[b], gsem.at[b]).wait()
            pltpu.make_async_copy(ndv.at[di], svals_v.at[pl.ds(b * 16, 16)],
                                  g2sem.at[b]).wait()
            pltpu.sync_copy(rows_v.at[b], acc_sh.at[di], add=True)
            pltpu.sync_copy(svals_v.at[pl.ds(b * 16, 16)], s_sh.at[si],
                            add=True)

            @pl.when(sc < nsuper - 1)
            def _():
                fire_gather(i + RING, b)

        return carry

    lax.fori_loop(0, nsuper, body, 0)
    plsc.subcore_barrier()

    @pl.when(c == 0)
    def _():
        pltpu.sync_copy(acc_sh.at[pl.ds(r0, ROWS_PER_TILE)],
                        agg_parts.at[0].at[pl.ds(r0, ROWS_PER_TILE)])

        @pl.when(s == 0)
        def _():
            pltpu.sync_copy(s_sh, s_parts.at[0])

    @pl.when(c == 1)
    def _():
        pltpu.sync_copy(acc_sh.at[pl.ds(r0, ROWS_PER_TILE)],
                        agg_parts.at[1].at[pl.ds(r0, ROWS_PER_TILE)])

        @pl.when(s == 0)
        def _():
            pltpu.sync_copy(s_sh, s_parts.at[1])


@functools.cache
def _agg_call():
    return pl.kernel(
        _agg_body,
        out_type=(
            jax.ShapeDtypeStruct((NC, NPAD, D_HID), jnp.float32),
            jax.ShapeDtypeStruct((NC, NPAD), jnp.float32),
        ),
        mesh=_sc_mesh(),
        scratch_types=[
            pltpu.VMEM_SHARED((NPAD, D_HID), jnp.float32),
            pltpu.VMEM_SHARED((NPAD,), jnp.float32),
            pltpu.VMEM((EDGES_W,), jnp.int32),
            pltpu.VMEM((EDGES_W,), jnp.int32),
            pltpu.VMEM((RING, 16, D_HID), jnp.float32),
            pltpu.VMEM((RING * 16,), jnp.float32),
            pltpu.SemaphoreType.DMA((RING,)),
            pltpu.SemaphoreType.DMA((RING,)),
        ],
    )


def _final_body(ap, ns, nd, sp, b1, w2, b2, wr, br, out_ref):
    agg = ap[0] + ap[1]
    x1 = jnp.maximum(agg * nd[...] + b1[...][None, :], 0.0)
    cvec = ns[...] * (sp[0] + sp[1])
    rid = lax.broadcasted_iota(jnp.int32, (NPAD, 1), 0)
    cvec = jnp.where(rid < N_NODES, cvec, 0.0)
    v = jnp.sum(x1 * cvec, axis=0, keepdims=True) * (1.0 / N_NODES)
    h2 = jnp.dot(v, w2[...], preferred_element_type=jnp.float32)
    h2 = h2 + b2[...][None, :]
    out = jnp.dot(h2, wr[...], preferred_element_type=jnp.float32)
    out_ref[...] = out + br[...][None, :]


_final_call = pl.pallas_call(
    _final_body,
    out_shape=jax.ShapeDtypeStruct((1, D_OUT), jnp.float32),
)


def kernel(features, edge_index, W1, b1, W2, b2, Wr, br):
    src = edge_index[0].astype(jnp.int32)
    dst = edge_index[1].astype(jnp.int32)
    pad = PADID + (jnp.arange(EPAD - N_EDGES, dtype=jnp.int32)
                   % (NPAD - N_NODES))
    src1 = jnp.concatenate([src, pad])
    dst1 = jnp.concatenate([dst, pad])
    srcp = src1.reshape(NGRP, GSZ)
    dstp = dst1.reshape(NGRP, GSZ)
    z1 = jnp.zeros((NPAD,), jnp.float32)
    z2 = jnp.zeros((NPAD, D_HID), jnp.float32)
    onesg = jnp.ones((GSZ,), jnp.float32)

    dego_p, degi_p = _deg_call()(srcp, dstp, onesg, z1)
    g, ns2, nd2 = _prep_call(features, W1,
                             dego_p[:, :, None], degi_p[:, :, None])
    agg_p, s_p = _agg_call()(g, src1, dst1, nd2.reshape(NPAD), z2, z1)
    out = _final_call(agg_p, ns2, nd2, s_p[:, :, None], b1, W2, b2, Wr, br)
    return out


# speedup vs baseline: 25.6906x; 1.0475x over previous
import functools

import jax
import jax.numpy as jnp
from jax import lax
from jax.experimental import pallas as pl
from jax.experimental.pallas import tpu as pltpu
from jax.experimental.pallas import tpu_sc as plsc

N_NODES = 10000
N_EDGES = 320000
D_IN = 128
D_HID = 128
D_OUT = 16

NPAD = 10240
PADID = N_NODES
EPAD = 327680
GSZ = 128
NGRP = EPAD // GSZ

NC, NS = 2, 16
ROWS_PER_TILE = NPAD // NS
GRP_W = NGRP // (NC * NS)
EDGES_W = EPAD // (NC * NS)
STEPS = EDGES_W // 16


@functools.cache
def _sc_mesh():
    return plsc.VectorSubcoreMesh(core_axis_name="c", subcore_axis_name="s",
                                  num_cores=NC, num_subcores=NS)


def _deg_body(srcp, dstp, ones_h, z1d, dego_out, degi_out,
              dego_sh, degi_sh, src_v, dst_v, ones_v):
    c = lax.axis_index("c")
    s = lax.axis_index("s")
    w = c * NS + s
    r0 = s * ROWS_PER_TILE
    pltpu.sync_copy(z1d.at[pl.ds(r0, ROWS_PER_TILE)],
                    dego_sh.at[pl.ds(r0, ROWS_PER_TILE)])
    pltpu.sync_copy(z1d.at[pl.ds(r0, ROWS_PER_TILE)],
                    degi_sh.at[pl.ds(r0, ROWS_PER_TILE)])
    pltpu.sync_copy(ones_h, ones_v)
    pltpu.sync_copy(srcp.at[pl.ds(w * GRP_W, GRP_W)], src_v)
    pltpu.sync_copy(dstp.at[pl.ds(w * GRP_W, GRP_W)], dst_v)
    plsc.subcore_barrier()

    def body(j, carry):
        pltpu.sync_copy(ones_v, dego_sh.at[src_v.at[j]], add=True)
        pltpu.sync_copy(ones_v, degi_sh.at[dst_v.at[j]], add=True)
        return carry

    lax.fori_loop(0, GRP_W, body, 0)
    plsc.subcore_barrier()

    @pl.when(s == 0)
    def _():
        pltpu.sync_copy(dego_sh, dego_out.at[c])
        pltpu.sync_copy(degi_sh, degi_out.at[c])


@functools.cache
def _deg_call():
    return pl.kernel(
        _deg_body,
        out_type=(
            jax.ShapeDtypeStruct((NC, NPAD), jnp.float32),
            jax.ShapeDtypeStruct((NC, NPAD), jnp.float32),
        ),
        mesh=_sc_mesh(),
        scratch_types=[
            pltpu.VMEM_SHARED((NPAD,), jnp.float32),
            pltpu.VMEM_SHARED((NPAD,), jnp.float32),
            pltpu.VMEM((GRP_W, GSZ), jnp.int32),
            pltpu.VMEM((GRP_W, GSZ), jnp.int32),
            pltpu.VMEM((GSZ,), jnp.float32),
        ],
    )


def _prep_body(x_ref, w1_ref, dego_ref, degi_ref, g_ref, ns_ref, nd_ref):
    dego = dego_ref[0] + dego_ref[1]
    degi = degi_ref[0] + degi_ref[1]
    ns = lax.rsqrt(jnp.where(dego > 0.0, dego, 1.0))
    nd = lax.rsqrt(jnp.where(degi > 0.0, degi, 1.0))
    h = jnp.dot(x_ref[...], w1_ref[...], preferred_element_type=jnp.float32)
    g_ref[pl.ds(0, N_NODES), :] = h * ns[:N_NODES]
    g_ref[pl.ds(N_NODES, NPAD - N_NODES), :] = jnp.zeros(
        (NPAD - N_NODES, D_HID), jnp.float32)
    ns_ref[...] = ns
    nd_ref[...] = nd


_prep_call = pl.pallas_call(
    _prep_body,
    out_shape=(
        jax.ShapeDtypeStruct((NPAD, D_HID), jnp.float32),
        jax.ShapeDtypeStruct((NPAD, 1), jnp.float32),
        jax.ShapeDtypeStruct((NPAD, 1), jnp.float32),
    ),
)


NG_W = EDGES_W // 128
NSUP = NG_W // 4


def _agg_body(g, srcp, dstp, ndv, z2d, z1d,
              agg_parts, s_parts,
              acc_sh, s_sh, src_v, dst_v, rows2, svals,
              isems, isemd, rgsem, sgsem):
    c = lax.axis_index("c")
    s = lax.axis_index("s")
    w = c * NS + s
    base = w * NG_W
    r0 = s * ROWS_PER_TILE
    pltpu.sync_copy(z2d.at[pl.ds(r0, ROWS_PER_TILE)],
                    acc_sh.at[pl.ds(r0, ROWS_PER_TILE)])

    @pl.when(s == 0)
    def _():
        pltpu.sync_copy(z1d, s_sh)

    plsc.subcore_barrier()

    def fetch_idx(j, b):
        pltpu.async_copy(srcp.at[base + j], src_v.at[b], isems.at[b])
        pltpu.async_copy(dstp.at[base + j], dst_v.at[b], isemd.at[b])

    def fire_gathers(j, b4, br):
        pltpu.make_async_copy(srcp.at[base + j], src_v.at[b4],
                              isems.at[b4]).wait()
        pltpu.make_async_copy(dstp.at[base + j], dst_v.at[b4],
                              isemd.at[b4]).wait()
        pltpu.async_copy(g.at[src_v.at[b4]], rows2.at[br], rgsem.at[br])
        pltpu.async_copy(ndv.at[dst_v.at[b4]], svals.at[br], sgsem.at[br])

    for k in range(3):
        fetch_idx(k, k)
    fire_gathers(0, 0, 0)

    def body(jj, carry):
        for b4 in range(4):
            j = jj * 4 + b4
            br = b4 % 2

            if b4 == 0:
                fetch_idx(j + 3, (b4 + 3) % 4)
            else:
                @pl.when(jj < NSUP - 1)
                def _():
                    fetch_idx(j + 3, (b4 + 3) % 4)

            pltpu.make_async_copy(g.at[src_v.at[b4]], rows2.at[br],
                                  rgsem.at[br]).wait()
            pltpu.make_async_copy(ndv.at[dst_v.at[b4]], svals.at[br],
                                  sgsem.at[br]).wait()

            if b4 == 3:
                @pl.when(jj < NSUP - 1)
                def _():
                    fire_gathers(j + 1, 0, (br + 1) % 2)
            else:
                fire_gathers(j + 1, b4 + 1, (br + 1) % 2)

            pltpu.sync_copy(rows2.at[br], acc_sh.at[dst_v.at[b4]], add=True)
            pltpu.sync_copy(svals.at[br], s_sh.at[src_v.at[b4]], add=True)

        return carry

    lax.fori_loop(0, NSUP, body, 0)
    plsc.subcore_barrier()

    @pl.when(c == 0)
    def _():
        pltpu.sync_copy(acc_sh.at[pl.ds(r0, ROWS_PER_TILE)],
                        agg_parts.at[0].at[pl.ds(r0, ROWS_PER_TILE)])

        @pl.when(s == 0)
        def _():
            pltpu.sync_copy(s_sh, s_parts.at[0])

    @pl.when(c == 1)
    def _():
        pltpu.sync_copy(acc_sh.at[pl.ds(r0, ROWS_PER_TILE)],
                        agg_parts.at[1].at[pl.ds(r0, ROWS_PER_TILE)])

        @pl.when(s == 0)
        def _():
            pltpu.sync_copy(s_sh, s_parts.at[1])


@functools.cache
def _agg_call():
    return pl.kernel(
        _agg_body,
        out_type=(
            jax.ShapeDtypeStruct((NC, NPAD, D_HID), jnp.float32),
            jax.ShapeDtypeStruct((NC, NPAD), jnp.float32),
        ),
        mesh=_sc_mesh(),
        scratch_types=[
            pltpu.VMEM_SHARED((NPAD, D_HID), jnp.float32),
            pltpu.VMEM_SHARED((NPAD,), jnp.float32),
            pltpu.VMEM((4, 128), jnp.int32),
            pltpu.VMEM((4, 128), jnp.int32),
            pltpu.VMEM((2, 128, D_HID), jnp.float32),
            pltpu.VMEM((2, 128), jnp.float32),
            pltpu.SemaphoreType.DMA((4,)),
            pltpu.SemaphoreType.DMA((4,)),
            pltpu.SemaphoreType.DMA((2,)),
            pltpu.SemaphoreType.DMA((2,)),
        ],
    )


def _final_body(ap, ns, nd, sp, b1, w2, b2, wr, br, out_ref):
    agg = ap[0] + ap[1]
    x1 = jnp.maximum(agg * nd[...] + b1[...][None, :], 0.0)
    cvec = ns[...] * (sp[0] + sp[1])
    rid = lax.broadcasted_iota(jnp.int32, (NPAD, 1), 0)
    cvec = jnp.where(rid < N_NODES, cvec, 0.0)
    v = jnp.sum(x1 * cvec, axis=0, keepdims=True) * (1.0 / N_NODES)
    h2 = jnp.dot(v, w2[...], preferred_element_type=jnp.float32)
    h2 = h2 + b2[...][None, :]
    out = jnp.dot(h2, wr[...], preferred_element_type=jnp.float32)
    out_ref[...] = out + br[...][None, :]


_final_call = pl.pallas_call(
    _final_body,
    out_shape=jax.ShapeDtypeStruct((1, D_OUT), jnp.float32),
)


def kernel(features, edge_index, W1, b1, W2, b2, Wr, br):
    src = edge_index[0].astype(jnp.int32)
    dst = edge_index[1].astype(jnp.int32)
    pad = PADID + (jnp.arange(EPAD - N_EDGES, dtype=jnp.int32)
                   % (NPAD - N_NODES))
    src1 = jnp.concatenate([src, pad])
    dst1 = jnp.concatenate([dst, pad])
    srcp = src1.reshape(NGRP, GSZ)
    dstp = dst1.reshape(NGRP, GSZ)
    z1 = jnp.zeros((NPAD,), jnp.float32)
    z2 = jnp.zeros((NPAD, D_HID), jnp.float32)
    onesg = jnp.ones((GSZ,), jnp.float32)

    dego_p, degi_p = _deg_call()(srcp, dstp, onesg, z1)
    g, ns2, nd2 = _prep_call(features, W1,
                             dego_p[:, :, None], degi_p[:, :, None])
    agg_p, s_p = _agg_call()(g, srcp, dstp, nd2.reshape(NPAD), z2, z1)
    out = _final_call(agg_p, ns2, nd2, s_p[:, :, None], b1, W2, b2, Wr, br)
    return out
